# baseline (speedup 1.0000x reference)
import functools

import jax
import jax.numpy as jnp
from jax import lax
from jax.experimental import pallas as pl
from jax.experimental.pallas import tpu as pltpu
from jax.experimental.pallas import tpu_sc as plsc

_MAX_DIST = 1000
_NSHIFT = 16
_NUM_CORES = 2
_NUM_SUBCORES = 16
_H_SC = 4


@functools.lru_cache(maxsize=None)
def _build_sc_kernel(S, H, H_SC, VLEN):
    NW = _NUM_CORES * _NUM_SUBCORES
    RPW = H_SC * S // NW
    HEAD0 = H - H_SC
    assert (H_SC * S) % NW == 0 and S % RPW == 0 and RPW % _NSHIFT == 0
    W = 2 * S
    assert VLEN >= W + _NSHIFT
    CH = 16

    mesh = plsc.VectorSubcoreMesh(
        core_axis_name="c", subcore_axis_name="s",
        num_cores=_NUM_CORES, num_subcores=_NUM_SUBCORES)

    @functools.partial(
        pl.kernel,
        out_type=jax.ShapeDtypeStruct((H * S * S,), jnp.float32),
        mesh=mesh,
        scratch_types=(
            [pltpu.VMEM((VLEN,), jnp.float32)]
            + [pltpu.VMEM((W,), jnp.float32)] * _NSHIFT
            + [pltpu.SemaphoreType.DMA]
        ),
    )
    def sc_kernel(v_hbm, out_hbm, vsrc, *rest):
        vs = rest[:_NSHIFT]
        sem = rest[_NSHIFT]
        wid = lax.axis_index("s") * _NUM_CORES + lax.axis_index("c")
        r0 = HEAD0 * S + wid * RPW
        h = r0 // S
        i0 = r0 - h * S

        pltpu.sync_copy(v_hbm.at[h], vsrc)

        for m in range(_NSHIFT):
            def shift_body(kc, _, m=m):
                vs[m][pl.ds(kc * CH, CH)] = vsrc[pl.ds(m + kc * CH, CH)]
                return _
            lax.fori_loop(0, W // CH, shift_body, None)

        for m in range(_NSHIFT):
            o = (S - 1 - m) % _NSHIFT

            def fire(t, _, m=m, o=o):
                i = i0 + o + t * _NSHIFT
                a = pl.multiple_of((S - 1) - i - m, _NSHIFT)
                pltpu.make_async_copy(
                    vs[m].at[pl.ds(a, S)],
                    out_hbm.at[pl.ds((r0 + o + t * _NSHIFT) * S, S)],
                    sem).start()
                return _
            lax.fori_loop(0, RPW // _NSHIFT, fire, None)

        def drain(t, _):
            pltpu.make_async_copy(
                vs[0].at[pl.ds(0, S)], out_hbm.at[pl.ds(r0 * S, S)],
                sem).wait()
            return _
        lax.fori_loop(0, RPW, drain, None)

    return sc_kernel


@functools.lru_cache(maxsize=None)
def _build_tc_kernel(S, H, H_TC, VLEN):
    NP = 128
    RB = 256
    WV = 2 * S
    assert S % RB == 0 and RB % 128 == 0
    assert VLEN >= WV + NP

    def body(buf_ref, v_ref, out_ref, varrev):
        del buf_ref
        b = pl.program_id(1)

        @pl.when(b == 0)
        def _build():
            for p in range(NP):
                src_off = 8 * (p // 8) + 7 - (p % 8)
                varrev[p, :] = v_ref[0, 0, pl.ds(src_off, WV)]

        for rg in range(RB // 8):
            m = (S - 8 - 8 * rg) % NP
            a128 = pl.multiple_of((S - 8 - 8 * rg - m) - b * RB, NP)
            out_ref[0, pl.ds(rg * 8, 8), :] = varrev[
                pl.ds(m, 8), pl.ds(a128, S)]

    return pl.pallas_call(
        body,
        grid=(H_TC, S // RB),
        in_specs=[
            pl.BlockSpec(memory_space=pl.ANY),
            pl.BlockSpec((1, 1, VLEN), lambda h, b: (h, 0, 0)),
        ],
        out_specs=pl.BlockSpec((1, RB, S), lambda h, b: (h, b, 0)),
        out_shape=jax.ShapeDtypeStruct((H, S, S), jnp.float32),
        scratch_shapes=[pltpu.VMEM((NP, WV), jnp.float32)],
        input_output_aliases={0: 0},
    )


def kernel(x, bias_values):
    S = x.shape[1]
    NB, H = bias_values.shape
    assert NB == 2 * _MAX_DIST
    VLEN = 2 * S + 128
    n_left = (S - 1) - _MAX_DIST
    n_right = VLEN - n_left - NB
    assert n_left >= 0 and n_right >= 1

    v = jnp.concatenate([
        jnp.broadcast_to(bias_values[0], (n_left, H)),
        bias_values,
        jnp.broadcast_to(bias_values[-1], (n_right, H)),
    ], axis=0).T

    h_tc = H - _H_SC
    buf = _build_sc_kernel(S, H, _H_SC, VLEN)(v)
    out = _build_tc_kernel(S, H, h_tc, VLEN)(
        buf.reshape(H, S, S), v[:h_tc].reshape(h_tc, 1, VLEN))
    return out.reshape(1, H, S, S)


# speedup vs baseline: 1.7647x; 1.7647x over previous
import functools

import jax
import jax.numpy as jnp
from jax import lax
from jax.experimental import pallas as pl
from jax.experimental.pallas import tpu as pltpu
from jax.experimental.pallas import tpu_sc as plsc

_MAX_DIST = 1000
_NSHIFT = 16
_NUM_CORES = 2
_NUM_SUBCORES = 16
_H_SC = 4


@functools.lru_cache(maxsize=None)
def _build_sc_kernel(S, H, H_SC, VLEN):
    NW = _NUM_CORES * _NUM_SUBCORES
    RPW = H_SC * S // NW
    HEAD0 = H - H_SC
    assert (H_SC * S) % NW == 0 and S % RPW == 0 and RPW % _NSHIFT == 0
    W = 2 * S
    assert VLEN >= W + _NSHIFT
    CH = 16

    mesh = plsc.VectorSubcoreMesh(
        core_axis_name="c", subcore_axis_name="s",
        num_cores=_NUM_CORES, num_subcores=_NUM_SUBCORES)

    @functools.partial(
        pl.kernel,
        out_type=jax.ShapeDtypeStruct((H_SC * S * S,), jnp.float32),
        mesh=mesh,
        scratch_types=(
            [pltpu.VMEM((VLEN,), jnp.float32)]
            + [pltpu.VMEM((W,), jnp.float32)] * _NSHIFT
            + [pltpu.SemaphoreType.DMA]
        ),
    )
    def sc_kernel(v_hbm, out_hbm, vsrc, *rest):
        vs = rest[:_NSHIFT]
        sem = rest[_NSHIFT]
        wid = lax.axis_index("s") * _NUM_CORES + lax.axis_index("c")
        r0 = wid * RPW
        h = r0 // S
        i0 = r0 - h * S

        pltpu.sync_copy(v_hbm.at[HEAD0 + h], vsrc)

        for m in range(_NSHIFT):
            def shift_body(kc, _, m=m):
                vs[m][pl.ds(kc * CH, CH)] = vsrc[pl.ds(m + kc * CH, CH)]
                return _
            lax.fori_loop(0, W // CH, shift_body, None)

        for m in range(_NSHIFT):
            o = (S - 1 - m) % _NSHIFT

            def fire(t, _, m=m, o=o):
                i = i0 + o + t * _NSHIFT
                a = pl.multiple_of((S - 1) - i - m, _NSHIFT)
                pltpu.make_async_copy(
                    vs[m].at[pl.ds(a, S)],
                    out_hbm.at[pl.ds((r0 + o + t * _NSHIFT) * S, S)],
                    sem).start()
                return _
            lax.fori_loop(0, RPW // _NSHIFT, fire, None)

        def drain(t, _):
            pltpu.make_async_copy(
                vs[0].at[pl.ds(0, S)], out_hbm.at[pl.ds(r0 * S, S)],
                sem).wait()
            return _
        lax.fori_loop(0, RPW, drain, None)

    return sc_kernel


@functools.lru_cache(maxsize=None)
def _build_tc_kernel(S, H, H_TC, VLEN):
    NP = 128
    RB = 256
    WV = 2 * S
    assert S % RB == 0 and RB % 128 == 0
    assert VLEN >= WV + NP

    def body(v_ref, out_ref, varrev):
        b = pl.program_id(1)

        @pl.when(b == 0)
        def _build():
            for p in range(NP):
                src_off = 8 * (p // 8) + 7 - (p % 8)
                varrev[p, :] = v_ref[0, 0, pl.ds(src_off, WV)]

        for rg in range(RB // 8):
            m = (S - 8 - 8 * rg) % NP
            a128 = pl.multiple_of((S - 8 - 8 * rg - m) - b * RB, NP)
            out_ref[0, pl.ds(rg * 8, 8), :] = varrev[
                pl.ds(m, 8), pl.ds(a128, S)]

    return pl.pallas_call(
        body,
        grid=(H_TC, S // RB),
        in_specs=[
            pl.BlockSpec((1, 1, VLEN), lambda h, b: (h, 0, 0)),
        ],
        out_specs=pl.BlockSpec((1, RB, S), lambda h, b: (h, b, 0)),
        out_shape=jax.ShapeDtypeStruct((H, S, S), jnp.float32),
        scratch_shapes=[pltpu.VMEM((NP, WV), jnp.float32)],
    )


def kernel(x, bias_values):
    S = x.shape[1]
    NB, H = bias_values.shape
    assert NB == 2 * _MAX_DIST
    VLEN = 2 * S + 128
    n_left = (S - 1) - _MAX_DIST
    n_right = VLEN - n_left - NB
    assert n_left >= 0 and n_right >= 1

    v = jnp.concatenate([
        jnp.broadcast_to(bias_values[0], (n_left, H)),
        bias_values,
        jnp.broadcast_to(bias_values[-1], (n_right, H)),
    ], axis=0).T

    h_tc = H - _H_SC
    sc = _build_sc_kernel(S, H, _H_SC, VLEN)(v)
    tc = _build_tc_kernel(S, H, h_tc, VLEN)(v[:h_tc].reshape(h_tc, 1, VLEN))
    out = lax.dynamic_update_slice(
        tc, sc.reshape(_H_SC, S, S), (h_tc, 0, 0))
    return out.reshape(1, H, S, S)


# speedup vs baseline: 2.2054x; 1.2498x over previous
import functools

import jax
import jax.numpy as jnp
from jax import lax
from jax.experimental import pallas as pl
from jax.experimental.pallas import tpu as pltpu
from jax.experimental.pallas import tpu_sc as plsc

_MAX_DIST = 1000
_NSHIFT = 16
_NUM_CORES = 2
_NUM_SUBCORES = 16
_H_SC = 2


@functools.lru_cache(maxsize=None)
def _build_sc_kernel(S, H, H_SC, VLEN):
    NW = _NUM_CORES * _NUM_SUBCORES
    RPW = H_SC * S // NW
    HEAD0 = H - H_SC
    assert (H_SC * S) % NW == 0 and S % RPW == 0 and RPW % _NSHIFT == 0
    W = 2 * S
    assert VLEN >= W + _NSHIFT
    CH = 16

    mesh = plsc.VectorSubcoreMesh(
        core_axis_name="c", subcore_axis_name="s",
        num_cores=_NUM_CORES, num_subcores=_NUM_SUBCORES)

    @functools.partial(
        pl.kernel,
        out_type=jax.ShapeDtypeStruct((H_SC * S * S,), jnp.float32),
        mesh=mesh,
        scratch_types=(
            [pltpu.VMEM((VLEN,), jnp.float32)]
            + [pltpu.VMEM((W,), jnp.float32)] * _NSHIFT
            + [pltpu.SemaphoreType.DMA]
        ),
    )
    def sc_kernel(v_hbm, out_hbm, vsrc, *rest):
        vs = rest[:_NSHIFT]
        sem = rest[_NSHIFT]
        wid = lax.axis_index("s") * _NUM_CORES + lax.axis_index("c")
        r0 = wid * RPW
        h = r0 // S
        i0 = r0 - h * S

        pltpu.sync_copy(v_hbm.at[HEAD0 + h], vsrc)

        for m in range(_NSHIFT):
            def shift_body(kc, _, m=m):
                vs[m][pl.ds(kc * CH, CH)] = vsrc[pl.ds(m + kc * CH, CH)]
                return _
            lax.fori_loop(0, W // CH, shift_body, None)

        for m in range(_NSHIFT):
            o = (S - 1 - m) % _NSHIFT

            def fire(t, _, m=m, o=o):
                i = i0 + o + t * _NSHIFT
                a = pl.multiple_of((S - 1) - i - m, _NSHIFT)
                pltpu.make_async_copy(
                    vs[m].at[pl.ds(a, S)],
                    out_hbm.at[pl.ds((r0 + o + t * _NSHIFT) * S, S)],
                    sem).start()
                return _
            lax.fori_loop(0, RPW // _NSHIFT, fire, None)

        def drain(t, _):
            pltpu.make_async_copy(
                vs[0].at[pl.ds(0, S)], out_hbm.at[pl.ds(r0 * S, S)],
                sem).wait()
            return _
        lax.fori_loop(0, RPW, drain, None)

    return sc_kernel


@functools.lru_cache(maxsize=None)
def _build_tc_kernel(S, H, H_TC, VLEN):
    NP = 128
    RB = 256
    WV = 2 * S
    assert S % RB == 0 and RB % 128 == 0
    assert VLEN >= WV + NP

    def body(v_ref, out_ref, varrev):
        b = pl.program_id(1)

        @pl.when(b == 0)
        def _build():
            for p in range(NP):
                src_off = 8 * (p // 8) + 7 - (p % 8)
                varrev[p, :] = v_ref[0, 0, pl.ds(src_off, WV)]

        for rg in range(RB // 8):
            m = (S - 8 - 8 * rg) % NP
            a128 = pl.multiple_of((S - 8 - 8 * rg - m) - b * RB, NP)
            out_ref[0, pl.ds(rg * 8, 8), :] = varrev[
                pl.ds(m, 8), pl.ds(a128, S)]

    return pl.pallas_call(
        body,
        grid=(H_TC, S // RB),
        in_specs=[
            pl.BlockSpec((1, 1, VLEN), lambda h, b: (h, 0, 0)),
        ],
        out_specs=pl.BlockSpec((1, RB, S), lambda h, b: (h, b, 0)),
        out_shape=jax.ShapeDtypeStruct((H, S, S), jnp.float32),
        scratch_shapes=[pltpu.VMEM((NP, WV), jnp.float32)],
    )


def kernel(x, bias_values):
    S = x.shape[1]
    NB, H = bias_values.shape
    assert NB == 2 * _MAX_DIST
    VLEN = 2 * S + 128
    n_left = (S - 1) - _MAX_DIST
    n_right = VLEN - n_left - NB
    assert n_left >= 0 and n_right >= 1

    v = jnp.concatenate([
        jnp.broadcast_to(bias_values[0], (n_left, H)),
        bias_values,
        jnp.broadcast_to(bias_values[-1], (n_right, H)),
    ], axis=0).T

    h_tc = H - _H_SC
    sc = _build_sc_kernel(S, H, _H_SC, VLEN)(v)
    tc = _build_tc_kernel(S, H, h_tc, VLEN)(v[:h_tc].reshape(h_tc, 1, VLEN))
    out = lax.dynamic_update_slice(
        tc, sc.reshape(_H_SC, S, S), (h_tc, 0, 0))
    return out.reshape(1, H, S, S)


# speedup vs baseline: 2.5306x; 1.1475x over previous
import functools

import jax
import jax.numpy as jnp
from jax import lax
from jax.experimental import pallas as pl
from jax.experimental.pallas import tpu as pltpu
from jax.experimental.pallas import tpu_sc as plsc

_MAX_DIST = 1000
_NSHIFT = 16
_NUM_CORES = 2
_NUM_SUBCORES = 16
_H_SC = 1


@functools.lru_cache(maxsize=None)
def _build_sc_kernel(S, H, H_SC, VLEN):
    NW = _NUM_CORES * _NUM_SUBCORES
    RPW = H_SC * S // NW
    HEAD0 = H - H_SC
    assert (H_SC * S) % NW == 0 and S % RPW == 0 and RPW % _NSHIFT == 0
    W = 2 * S
    assert VLEN >= W + _NSHIFT
    CH = 16

    mesh = plsc.VectorSubcoreMesh(
        core_axis_name="c", subcore_axis_name="s",
        num_cores=_NUM_CORES, num_subcores=_NUM_SUBCORES)

    @functools.partial(
        pl.kernel,
        out_type=jax.ShapeDtypeStruct((H_SC * S * S,), jnp.float32),
        mesh=mesh,
        scratch_types=(
            [pltpu.VMEM((VLEN,), jnp.float32)]
            + [pltpu.VMEM((W,), jnp.float32)] * _NSHIFT
            + [pltpu.SemaphoreType.DMA]
        ),
    )
    def sc_kernel(v_hbm, out_hbm, vsrc, *rest):
        vs = rest[:_NSHIFT]
        sem = rest[_NSHIFT]
        wid = lax.axis_index("s") * _NUM_CORES + lax.axis_index("c")
        r0 = wid * RPW
        h = r0 // S
        i0 = r0 - h * S

        pltpu.sync_copy(v_hbm.at[HEAD0 + h], vsrc)

        for m in range(_NSHIFT):
            def shift_body(kc, _, m=m):
                vs[m][pl.ds(kc * CH, CH)] = vsrc[pl.ds(m + kc * CH, CH)]
                return _
            lax.fori_loop(0, W // CH, shift_body, None)

        for m in range(_NSHIFT):
            o = (S - 1 - m) % _NSHIFT

            def fire(t, _, m=m, o=o):
                i = i0 + o + t * _NSHIFT
                a = pl.multiple_of((S - 1) - i - m, _NSHIFT)
                pltpu.make_async_copy(
                    vs[m].at[pl.ds(a, S)],
                    out_hbm.at[pl.ds((r0 + o + t * _NSHIFT) * S, S)],
                    sem).start()
                return _
            lax.fori_loop(0, RPW // _NSHIFT, fire, None)

        def drain(t, _):
            pltpu.make_async_copy(
                vs[0].at[pl.ds(0, S)], out_hbm.at[pl.ds(r0 * S, S)],
                sem).wait()
            return _
        lax.fori_loop(0, RPW, drain, None)

    return sc_kernel


@functools.lru_cache(maxsize=None)
def _build_tc_kernel(S, H, H_TC, VLEN):
    NP = 128
    RB = 256
    WV = 2 * S
    assert S % RB == 0 and RB % 128 == 0
    assert VLEN >= WV + NP

    def body(v_ref, out_ref, varrev):
        b = pl.program_id(1)

        @pl.when(b == 0)
        def _build():
            for p in range(NP):
                src_off = 8 * (p // 8) + 7 - (p % 8)
                varrev[p, :] = v_ref[0, 0, pl.ds(src_off, WV)]

        for rg in range(RB // 8):
            m = (S - 8 - 8 * rg) % NP
            a128 = pl.multiple_of((S - 8 - 8 * rg - m) - b * RB, NP)
            out_ref[0, pl.ds(rg * 8, 8), :] = varrev[
                pl.ds(m, 8), pl.ds(a128, S)]

    return pl.pallas_call(
        body,
        grid=(H_TC, S // RB),
        in_specs=[
            pl.BlockSpec((1, 1, VLEN), lambda h, b: (h, 0, 0)),
        ],
        out_specs=pl.BlockSpec((1, RB, S), lambda h, b: (h, b, 0)),
        out_shape=jax.ShapeDtypeStruct((H, S, S), jnp.float32),
        scratch_shapes=[pltpu.VMEM((NP, WV), jnp.float32)],
    )


def kernel(x, bias_values):
    S = x.shape[1]
    NB, H = bias_values.shape
    assert NB == 2 * _MAX_DIST
    VLEN = 2 * S + 128
    n_left = (S - 1) - _MAX_DIST
    n_right = VLEN - n_left - NB
    assert n_left >= 0 and n_right >= 1

    v = jnp.concatenate([
        jnp.broadcast_to(bias_values[0], (n_left, H)),
        bias_values,
        jnp.broadcast_to(bias_values[-1], (n_right, H)),
    ], axis=0).T

    h_tc = H - _H_SC
    sc = _build_sc_kernel(S, H, _H_SC, VLEN)(v)
    tc = _build_tc_kernel(S, H, h_tc, VLEN)(v[:h_tc].reshape(h_tc, 1, VLEN))
    out = lax.dynamic_update_slice(
        tc, sc.reshape(_H_SC, S, S), (h_tc, 0, 0))
    return out.reshape(1, H, S, S)
